# baseline (speedup 1.0000x reference)
import functools

import jax
import jax.numpy as jnp
from jax import lax
from jax.experimental import pallas as pl
from jax.experimental.pallas import tpu as pltpu
from jax.experimental.pallas import tpu_sc as plsc

BATCH = 16384
EMBED_DIM = 32
HIDDEN_DIM = 64
_PACK = 4
_PROW = _PACK * EMBED_DIM

_NUM_CORES = 2
_NUM_SUBCORES = 16
_NW = _NUM_CORES * _NUM_SUBCORES
_B_PER_W = BATCH // _NW
_CHUNK = 128
_NCHUNK = _B_PER_W // _CHUNK


def _gather_body(uid_hbm, iid_hbm, up_hbm, ip_hbm, out_u, out_i,
                 idx_v, qidx, rows4, sem, isem):
    wid = lax.axis_index("s") * _NUM_CORES + lax.axis_index("c")
    base = wid * _B_PER_W

    def one_table(ids_hbm, packed_hbm, out_hbm):
        stage = []
        for j in range(_NCHUNK):
            stage.append(pltpu.async_copy(
                ids_hbm.at[pl.ds(base + j * _CHUNK, _CHUNK)],
                idx_v.at[j], isem))
        for c in stage:
            c.wait()
        for j in range(_NCHUNK):
            for v in range(_CHUNK // 16):
                r = idx_v[j, pl.ds(v * 16, 16)]
                qidx[j, pl.ds(v * 16, 16)] = lax.shift_right_logical(r, 2)
        copies = []
        for j in range(_NCHUNK):
            copies.append(pltpu.async_copy(
                packed_hbm.at[qidx.at[j]],
                rows4.at[pl.ds(j * _CHUNK, _CHUNK)], sem))
        for c in copies:
            c.wait()
        pltpu.sync_copy(rows4, out_hbm.at[pl.ds(base, _B_PER_W)])

    one_table(uid_hbm, up_hbm, out_u)
    one_table(iid_hbm, ip_hbm, out_i)


@functools.cache
def _sc_gather():
    return pl.kernel(
        _gather_body,
        out_type=(
            jax.ShapeDtypeStruct((BATCH, _PROW), jnp.float32),
            jax.ShapeDtypeStruct((BATCH, _PROW), jnp.float32),
        ),
        mesh=plsc.VectorSubcoreMesh(core_axis_name="c", subcore_axis_name="s"),
        scratch_types=[
            pltpu.VMEM((_NCHUNK, _CHUNK), jnp.int32),
            pltpu.VMEM((_NCHUNK, _CHUNK), jnp.int32),
            pltpu.VMEM((_B_PER_W, _PROW), jnp.float32),
            pltpu.SemaphoreType.DMA,
            pltpu.SemaphoreType.DMA,
        ],
    )


_MLP_BLK = 2048


def _mlp_body(yu_ref, yi_ref, ohu_ref, ohi_ref, mu_ref, mi_ref, b1_ref,
              w2_ref, b2_ref, out_ref):
    h = jnp.zeros((_MLP_BLK, HIDDEN_DIM), dtype=jnp.float32)
    yu = yu_ref[...]
    yi = yi_ref[...]
    for a in range(_PACK):
        hu = jnp.dot(yu, mu_ref[a], preferred_element_type=jnp.float32)
        hi = jnp.dot(yi, mi_ref[a], preferred_element_type=jnp.float32)
        h = h + hu * ohu_ref[:, a:a + 1] + hi * ohi_ref[:, a:a + 1]
    h = jnp.maximum(h + b1_ref[...], 0.0)
    y = jnp.sum(h * w2_ref[...], axis=1) + b2_ref[0, 0]
    out_ref[...] = jax.nn.sigmoid(y)


def _mlp_call(yu, yi, ohu, ohi, mu, mi, b1, w2, b2):
    grid = BATCH // _MLP_BLK
    return pl.pallas_call(
        _mlp_body,
        grid=(grid,),
        in_specs=[
            pl.BlockSpec((_MLP_BLK, _PROW), lambda i: (i, 0)),
            pl.BlockSpec((_MLP_BLK, _PROW), lambda i: (i, 0)),
            pl.BlockSpec((_MLP_BLK, _PACK), lambda i: (i, 0)),
            pl.BlockSpec((_MLP_BLK, _PACK), lambda i: (i, 0)),
            pl.BlockSpec((_PACK, _PROW, HIDDEN_DIM), lambda i: (0, 0, 0)),
            pl.BlockSpec((_PACK, _PROW, HIDDEN_DIM), lambda i: (0, 0, 0)),
            pl.BlockSpec((1, HIDDEN_DIM), lambda i: (0, 0)),
            pl.BlockSpec((1, HIDDEN_DIM), lambda i: (0, 0)),
            pl.BlockSpec((1, 1), lambda i: (0, 0)),
        ],
        out_specs=pl.BlockSpec((_MLP_BLK,), lambda i: (i,)),
        out_shape=jax.ShapeDtypeStruct((BATCH,), jnp.float32),
    )(yu, yi, ohu, ohi, mu, mi, b1, w2, b2)


def _shifted_weights(w_half):
    mats = []
    for a in range(_PACK):
        m = jnp.zeros((_PROW, HIDDEN_DIM), dtype=jnp.float32)
        m = lax.dynamic_update_slice(m, w_half, (a * EMBED_DIM, 0))
        mats.append(m)
    return jnp.stack(mats)


def kernel(user_ids, item_ids, user_emb, item_emb, W1, b1, W2, b2):
    n_users = user_emb.shape[0]
    n_items = item_emb.shape[0]
    up = user_emb.reshape(n_users // _PACK, _PROW)
    ip = item_emb.reshape(n_items // _PACK, _PROW)
    yu, yi = _sc_gather()(user_ids, item_ids, up, ip)
    ohu = jax.nn.one_hot(jnp.bitwise_and(user_ids, _PACK - 1), _PACK,
                         dtype=jnp.float32)
    ohi = jax.nn.one_hot(jnp.bitwise_and(item_ids, _PACK - 1), _PACK,
                         dtype=jnp.float32)
    mu = _shifted_weights(W1[:, :EMBED_DIM].T)
    mi = _shifted_weights(W1[:, EMBED_DIM:].T)
    b1r = b1.reshape(1, HIDDEN_DIM)
    w2r = W2.reshape(1, HIDDEN_DIM)
    b2r = b2.reshape(1, 1)
    return _mlp_call(yu, yi, ohu, ohi, mu, mi, b1r, w2r, b2r)


# speedup vs baseline: 1.0534x; 1.0534x over previous
import functools

import jax
import jax.numpy as jnp
from jax import lax
from jax.experimental import pallas as pl
from jax.experimental.pallas import tpu as pltpu
from jax.experimental.pallas import tpu_sc as plsc

BATCH = 16384
EMBED_DIM = 32
HIDDEN_DIM = 64
_PACK = 4
_PROW = _PACK * EMBED_DIM

_NUM_CORES = 2
_NUM_SUBCORES = 16
_NW = _NUM_CORES * _NUM_SUBCORES
_B_PER_W = BATCH // _NW
_CHUNK = 128
_NCHUNK = _B_PER_W // _CHUNK


def _gather_body(uid_hbm, iid_hbm, up_hbm, ip_hbm, out_u, out_i,
                 idx_v, rows4, sem, isem):
    wid = lax.axis_index("s") * _NUM_CORES + lax.axis_index("c")
    base = wid * _B_PER_W

    def one_table(ids_hbm, packed_hbm, out_hbm):
        stage = []
        for j in range(_NCHUNK):
            stage.append(pltpu.async_copy(
                ids_hbm.at[pl.ds(base + j * _CHUNK, _CHUNK)],
                idx_v.at[j], isem))
        for c in stage:
            c.wait()
        copies = []
        for j in range(_NCHUNK):
            copies.append(pltpu.async_copy(
                packed_hbm.at[idx_v.at[j]],
                rows4.at[pl.ds(j * _CHUNK, _CHUNK)], sem))
        for c in copies:
            c.wait()
        pltpu.sync_copy(rows4, out_hbm.at[pl.ds(base, _B_PER_W)])

    one_table(uid_hbm, up_hbm, out_u)
    one_table(iid_hbm, ip_hbm, out_i)


@functools.cache
def _sc_gather():
    return pl.kernel(
        _gather_body,
        out_type=(
            jax.ShapeDtypeStruct((BATCH, _PROW), jnp.float32),
            jax.ShapeDtypeStruct((BATCH, _PROW), jnp.float32),
        ),
        mesh=plsc.VectorSubcoreMesh(core_axis_name="c", subcore_axis_name="s"),
        scratch_types=[
            pltpu.VMEM((_NCHUNK, _CHUNK), jnp.int32),
            pltpu.VMEM((_B_PER_W, _PROW), jnp.float32),
            pltpu.SemaphoreType.DMA,
            pltpu.SemaphoreType.DMA,
        ],
    )


_TR_IN = 1024
_TR_GRID = 977
_NQ = _TR_GRID * 2 * 128


def _pack_one(x, eye, valid):
    tb = lax.dot_general(x, eye, (((0,), (0,)), ((), ())),
                         preferred_element_type=jnp.float32)
    tb = jnp.where(valid, tb, 0.0)
    blks = []
    for tl in range(2):
        blks.append(jnp.concatenate(
            [tb[512 * tl + 128 * a:512 * tl + 128 * (a + 1)]
             for a in range(_PACK)], axis=1))
    return jnp.concatenate(blks, axis=0)


def _pack_body(u_ref, i_ref, up_ref, ip_ref):
    eye = jnp.eye(EMBED_DIM, dtype=jnp.float32)
    i = pl.program_id(0)
    rows = i * _TR_IN + lax.broadcasted_iota(jnp.int32, (_TR_IN, EMBED_DIM), 0)
    valid = rows < 1000000
    up_ref[...] = _pack_one(u_ref[...], eye, valid)
    ip_ref[...] = _pack_one(i_ref[...], eye, valid)


def _pack_call(uet, iet):
    return pl.pallas_call(
        _pack_body,
        grid=(_TR_GRID,),
        in_specs=[
            pl.BlockSpec((EMBED_DIM, _TR_IN), lambda i: (0, i)),
            pl.BlockSpec((EMBED_DIM, _TR_IN), lambda i: (0, i)),
        ],
        out_specs=(pl.BlockSpec((256, _PROW), lambda i: (i, 0)),
                   pl.BlockSpec((256, _PROW), lambda i: (i, 0))),
        out_shape=(jax.ShapeDtypeStruct((_NQ, _PROW), jnp.float32),
                   jax.ShapeDtypeStruct((_NQ, _PROW), jnp.float32)),
    )(uet, iet)


_MLP_BLK = 2048


def _mlp_body(yu_ref, yi_ref, ohu_ref, ohi_ref, mu_ref, mi_ref, b1_ref,
              w2_ref, b2_ref, out_ref):
    h = jnp.zeros((_MLP_BLK, HIDDEN_DIM), dtype=jnp.float32)
    yu = yu_ref[...]
    yi = yi_ref[...]
    for a in range(_PACK):
        hu = jnp.dot(yu, mu_ref[a], preferred_element_type=jnp.float32)
        hi = jnp.dot(yi, mi_ref[a], preferred_element_type=jnp.float32)
        h = h + hu * ohu_ref[:, a:a + 1] + hi * ohi_ref[:, a:a + 1]
    h = jnp.maximum(h + b1_ref[...], 0.0)
    y = jnp.sum(h * w2_ref[...], axis=1) + b2_ref[0, 0]
    out_ref[...] = jax.nn.sigmoid(y)


def _mlp_call(yu, yi, ohu, ohi, mu, mi, b1, w2, b2):
    grid = BATCH // _MLP_BLK
    return pl.pallas_call(
        _mlp_body,
        grid=(grid,),
        in_specs=[
            pl.BlockSpec((_MLP_BLK, _PROW), lambda i: (i, 0)),
            pl.BlockSpec((_MLP_BLK, _PROW), lambda i: (i, 0)),
            pl.BlockSpec((_MLP_BLK, _PACK), lambda i: (i, 0)),
            pl.BlockSpec((_MLP_BLK, _PACK), lambda i: (i, 0)),
            pl.BlockSpec((_PACK, _PROW, HIDDEN_DIM), lambda i: (0, 0, 0)),
            pl.BlockSpec((_PACK, _PROW, HIDDEN_DIM), lambda i: (0, 0, 0)),
            pl.BlockSpec((1, HIDDEN_DIM), lambda i: (0, 0)),
            pl.BlockSpec((1, HIDDEN_DIM), lambda i: (0, 0)),
            pl.BlockSpec((1, 1), lambda i: (0, 0)),
        ],
        out_specs=pl.BlockSpec((_MLP_BLK,), lambda i: (i,)),
        out_shape=jax.ShapeDtypeStruct((BATCH,), jnp.float32),
    )(yu, yi, ohu, ohi, mu, mi, b1, w2, b2)


def _shifted_weights(w_half):
    mats = []
    for a in range(_PACK):
        m = jnp.zeros((_PROW, HIDDEN_DIM), dtype=jnp.float32)
        m = lax.dynamic_update_slice(m, w_half, (a * EMBED_DIM, 0))
        mats.append(m)
    return jnp.stack(mats)


def kernel(user_ids, item_ids, user_emb, item_emb, W1, b1, W2, b2):
    uet = user_emb.T
    iet = item_emb.T
    up, ip = _pack_call(uet, iet)
    qu = jnp.bitwise_or(jnp.left_shift(jnp.right_shift(user_ids, 9), 7),
                        jnp.bitwise_and(user_ids, 127))
    qi = jnp.bitwise_or(jnp.left_shift(jnp.right_shift(item_ids, 9), 7),
                        jnp.bitwise_and(item_ids, 127))
    yu, yi = _sc_gather()(qu, qi, up, ip)
    au = jnp.bitwise_and(jnp.right_shift(user_ids, 7), _PACK - 1)
    ai = jnp.bitwise_and(jnp.right_shift(item_ids, 7), _PACK - 1)
    ohu = jax.nn.one_hot(au, _PACK, dtype=jnp.float32)
    ohi = jax.nn.one_hot(ai, _PACK, dtype=jnp.float32)
    mu = _shifted_weights(W1[:, :EMBED_DIM].T)
    mi = _shifted_weights(W1[:, EMBED_DIM:].T)
    b1r = b1.reshape(1, HIDDEN_DIM)
    w2r = W2.reshape(1, HIDDEN_DIM)
    b2r = b2.reshape(1, 1)
    return _mlp_call(yu, yi, ohu, ohi, mu, mi, b1r, w2r, b2r)


# speedup vs baseline: 1.4401x; 1.3671x over previous
import functools

import jax
import jax.numpy as jnp
from jax import lax
from jax.experimental import pallas as pl
from jax.experimental.pallas import tpu as pltpu
from jax.experimental.pallas import tpu_sc as plsc

BATCH = 16384
EMBED_DIM = 32
HIDDEN_DIM = 64
_PACK = 4
_PROW = _PACK * EMBED_DIM

_NUM_CORES = 2
_NUM_SUBCORES = 16
_NW = _NUM_CORES * _NUM_SUBCORES
_B_PER_W = BATCH // _NW
_CHUNK = 128
_NCHUNK = _B_PER_W // _CHUNK


def _gather_body(uid_hbm, iid_hbm, up_hbm, ip_hbm, uet3_hbm, out_u, out_i,
                 idx_v, rows4, slab, sem, isem):
    wid = lax.axis_index("s") * _NUM_CORES + lax.axis_index("c")
    base = wid * _B_PER_W
    pltpu.async_copy(uet3_hbm.at[:, :, pl.ds(wid * 512, 512)], slab,
                     isem).wait()

    def one_table(ids_hbm, packed_hbm, out_hbm):
        stage = []
        for j in range(_NCHUNK):
            stage.append(pltpu.async_copy(
                ids_hbm.at[pl.ds(base + j * _CHUNK, _CHUNK)],
                idx_v.at[j], isem))
        for c in stage:
            c.wait()
        copies = []
        for j in range(_NCHUNK):
            copies.append(pltpu.async_copy(
                packed_hbm.at[idx_v.at[j]],
                rows4.at[pl.ds(j * _CHUNK, _CHUNK)], sem))
        for c in copies:
            c.wait()
        pltpu.sync_copy(rows4, out_hbm.at[pl.ds(base, _B_PER_W)])

    one_table(uid_hbm, up_hbm, out_u)
    one_table(iid_hbm, ip_hbm, out_i)


@functools.cache
def _sc_gather():
    return pl.kernel(
        _gather_body,
        out_type=(
            jax.ShapeDtypeStruct((BATCH, _PROW), jnp.float32),
            jax.ShapeDtypeStruct((BATCH, _PROW), jnp.float32),
        ),
        mesh=plsc.VectorSubcoreMesh(core_axis_name="c", subcore_axis_name="s"),
        scratch_types=[
            pltpu.VMEM((_NCHUNK, _CHUNK), jnp.int32),
            pltpu.VMEM((_B_PER_W, _PROW), jnp.float32),
            pltpu.VMEM((_PACK, 8, 512), jnp.float32),
            pltpu.SemaphoreType.DMA,
            pltpu.SemaphoreType.DMA,
        ],
    )


_TR_IN = 2048
_TR_GRID = 489
_TR_OUT = _TR_IN // _PACK
_NQ = _TR_GRID * _TR_OUT


def _pack_one(x):
    tb = x.T
    blks = []
    for tl in range(_TR_IN // 512):
        blks.append(jnp.concatenate(
            [tb[512 * tl + 128 * a:512 * tl + 128 * (a + 1)]
             for a in range(_PACK)], axis=1))
    return jnp.concatenate(blks, axis=0)


def _pack_body(u_ref, i_ref, up_ref, ip_ref):
    i = pl.program_id(0)
    pu = _pack_one(u_ref[...])
    pi = _pack_one(i_ref[...])
    up_ref[...] = pu
    ip_ref[...] = pi

    @pl.when(i == _TR_GRID - 1)
    def _():
        w = lax.broadcasted_iota(jnp.int32, (_TR_OUT, _PROW), 0)
        col = lax.broadcasted_iota(jnp.int32, (_TR_OUT, _PROW), 1)
        r_local = (512 * (w // 128) + 128 * (col // 32) + w % 128)
        valid = (i * _TR_IN + r_local) < 1000000
        up_ref[...] = jnp.where(valid, pu, 0.0)
        ip_ref[...] = jnp.where(valid, pi, 0.0)


def _pack_call(uet, iet):
    return pl.pallas_call(
        _pack_body,
        grid=(_TR_GRID,),
        in_specs=[
            pl.BlockSpec((EMBED_DIM, _TR_IN), lambda i: (0, i)),
            pl.BlockSpec((EMBED_DIM, _TR_IN), lambda i: (0, i)),
        ],
        out_specs=(pl.BlockSpec((_TR_OUT, _PROW), lambda i: (i, 0)),
                   pl.BlockSpec((_TR_OUT, _PROW), lambda i: (i, 0))),
        out_shape=(jax.ShapeDtypeStruct((_NQ, _PROW), jnp.float32),
                   jax.ShapeDtypeStruct((_NQ, _PROW), jnp.float32)),
    )(uet, iet)


_MLP_BLK = 2048


def _mlp_body(yu_ref, yi_ref, ohu_ref, ohi_ref, mu_ref, mi_ref, b1_ref,
              w2_ref, b2_ref, out_ref):
    h = jnp.zeros((_MLP_BLK, HIDDEN_DIM), dtype=jnp.float32)
    yu = yu_ref[...]
    yi = yi_ref[...]
    for a in range(_PACK):
        hu = jnp.dot(yu, mu_ref[a], preferred_element_type=jnp.float32)
        hi = jnp.dot(yi, mi_ref[a], preferred_element_type=jnp.float32)
        h = h + hu * ohu_ref[:, a:a + 1] + hi * ohi_ref[:, a:a + 1]
    h = jnp.maximum(h + b1_ref[...], 0.0)
    y = jnp.sum(h * w2_ref[...], axis=1) + b2_ref[0, 0]
    out_ref[...] = jax.nn.sigmoid(y)


def _mlp_call(yu, yi, ohu, ohi, mu, mi, b1, w2, b2):
    grid = BATCH // _MLP_BLK
    return pl.pallas_call(
        _mlp_body,
        grid=(grid,),
        in_specs=[
            pl.BlockSpec((_MLP_BLK, _PROW), lambda i: (i, 0)),
            pl.BlockSpec((_MLP_BLK, _PROW), lambda i: (i, 0)),
            pl.BlockSpec((_MLP_BLK, _PACK), lambda i: (i, 0)),
            pl.BlockSpec((_MLP_BLK, _PACK), lambda i: (i, 0)),
            pl.BlockSpec((_PACK, _PROW, HIDDEN_DIM), lambda i: (0, 0, 0)),
            pl.BlockSpec((_PACK, _PROW, HIDDEN_DIM), lambda i: (0, 0, 0)),
            pl.BlockSpec((1, HIDDEN_DIM), lambda i: (0, 0)),
            pl.BlockSpec((1, HIDDEN_DIM), lambda i: (0, 0)),
            pl.BlockSpec((1, 1), lambda i: (0, 0)),
        ],
        out_specs=pl.BlockSpec((_MLP_BLK,), lambda i: (i,)),
        out_shape=jax.ShapeDtypeStruct((BATCH,), jnp.float32),
    )(yu, yi, ohu, ohi, mu, mi, b1, w2, b2)


def _shifted_weights(w_half):
    mats = []
    for a in range(_PACK):
        m = jnp.zeros((_PROW, HIDDEN_DIM), dtype=jnp.float32)
        m = lax.dynamic_update_slice(m, w_half, (a * EMBED_DIM, 0))
        mats.append(m)
    return jnp.stack(mats)


def kernel(user_ids, item_ids, user_emb, item_emb, W1, b1, W2, b2):
    uet = user_emb.T
    iet = item_emb.T
    up, ip = _pack_call(uet, iet)
    qu = jnp.bitwise_or(jnp.left_shift(jnp.right_shift(user_ids, 9), 7),
                        jnp.bitwise_and(user_ids, 127))
    qi = jnp.bitwise_or(jnp.left_shift(jnp.right_shift(item_ids, 9), 7),
                        jnp.bitwise_and(item_ids, 127))
    uet3 = user_emb.T.reshape(_PACK, 8, user_emb.shape[0])
    yu, yi = _sc_gather()(qu, qi, up, ip, uet3)
    au = jnp.bitwise_and(jnp.right_shift(user_ids, 7), _PACK - 1)
    ai = jnp.bitwise_and(jnp.right_shift(item_ids, 7), _PACK - 1)
    ohu = jax.nn.one_hot(au, _PACK, dtype=jnp.float32)
    ohi = jax.nn.one_hot(ai, _PACK, dtype=jnp.float32)
    mu = _shifted_weights(W1[:, :EMBED_DIM].T)
    mi = _shifted_weights(W1[:, EMBED_DIM:].T)
    b1r = b1.reshape(1, HIDDEN_DIM)
    w2r = W2.reshape(1, HIDDEN_DIM)
    b2r = b2.reshape(1, 1)
    return _mlp_call(yu, yi, ohu, ohi, mu, mi, b1r, w2r, b2r)
